# baseline (speedup 1.0000x reference)
import functools

import jax
import jax.numpy as jnp
from jax import lax
from jax.experimental import pallas as pl
from jax.experimental.pallas import tpu as pltpu
from jax.experimental.pallas import tpu_sc as plsc

_B = 16384
_E = 64
_NUM_NEG = 5
_VOCAB = 1000000
_NJ = _NUM_NEG + 1
_NW = 32
_BPW = _B // _NW
_CHUNK = 16
_NCHUNK = _BPW // _CHUNK
_LANES = 16
_NG = _CHUNK // _LANES
_DUNROLL = 16
_BROWS = _NJ * _CHUNK + _CHUNK

_mesh = plsc.VectorSubcoreMesh(core_axis_name="c", subcore_axis_name="s")


@functools.partial(
    pl.kernel,
    mesh=_mesh,
    out_type=jax.ShapeDtypeStruct((_NJ * _B,), jnp.float32),
    scratch_types=[
        pltpu.VMEM((_NJ, _BPW), jnp.int32),
        pltpu.VMEM((_BROWS, _E), jnp.float32),
        pltpu.VMEM((_BROWS, _E), jnp.float32),
        pltpu.VMEM((_BROWS, _E), jnp.float32),
        pltpu.VMEM((_BROWS, _E), jnp.float32),
        pltpu.VMEM((_NJ, _BPW), jnp.float32),
        pltpu.SemaphoreType.DMA,
        pltpu.SemaphoreType.DMA,
        pltpu.SemaphoreType.DMA,
        pltpu.SemaphoreType.DMA,
    ],
    compiler_params=pltpu.CompilerParams(
        needs_layout_passes=False, use_tc_tiling_on_sc=True
    ),
)
def _sc_dots(hid_hbm, idx_hbm, table3_hbm, out_hbm,
             idx_v, buf_a, buf_b, buf_c, buf_d, dots_v,
             sem_a, sem_b, sem_c, sem_d):
    table_hbm = table3_hbm.at[0]
    wid = lax.axis_index("s") * 2 + lax.axis_index("c")
    base = wid * _BPW

    for j in range(_NJ):
        pltpu.sync_copy(idx_hbm.at[pl.ds(j * _B + base, _BPW)], idx_v.at[j])

    lane = lax.iota(jnp.int32, _LANES)
    buf_rows = [
        [lane + (j * _CHUNK + g * _LANES) for g in range(_NG)]
        for j in range(_NJ)
    ]
    hid_rows = [lane + (_NJ * _CHUNK + g * _LANES) for g in range(_NG)]
    zero = jnp.zeros((_LANES,), jnp.float32)

    def transfers(c, buf, sem):
        coff = c * _CHUNK
        for j in range(_NJ):
            for g in range(_NG):
                v = idx_v[j, pl.ds(coff + g * _LANES, _LANES)]
                for t in range(_LANES):
                    row = j * _CHUNK + g * _LANES + t
                    pltpu.async_copy(
                        table_hbm.at[pl.ds(v[t], 1), :],
                        buf.at[pl.ds(row, 1), :],
                        sem,
                    )
        pltpu.async_copy(
            hid_hbm.at[pl.ds(base + coff, _CHUNK), :],
            buf.at[pl.ds(_NJ * _CHUNK, _CHUNK), :],
            sem,
        )

    def drain(buf, sem):
        pltpu.make_async_copy(
            hid_hbm.at[pl.ds(0, _BROWS), :], buf, sem
        ).wait()

    def compute(c, buf):
        coff = c * _CHUNK
        for g in range(_NG):
            hrow = hid_rows[g]

            def dblk(k, accs, hrow=hrow, g=g, buf=buf):
                res = list(accs)
                d0 = k * _DUNROLL
                for t in range(_DUNROLL):
                    col = jnp.full((_LANES,), d0 + t, jnp.int32)
                    h = plsc.load_gather(buf, [hrow, col])
                    for j in range(_NJ):
                        e = plsc.load_gather(buf, [buf_rows[j][g], col])
                        res[j] = res[j] + e * h
                return tuple(res)

            accs = lax.fori_loop(0, _E // _DUNROLL, dblk, (zero,) * _NJ)
            for j in range(_NJ):
                dots_v[j, pl.ds(coff + g * _LANES, _LANES)] = accs[j]

    ring = [(buf_a, sem_a), (buf_b, sem_b), (buf_c, sem_c), (buf_d, sem_d)]
    for b in range(3):
        transfers(b, *ring[b])

    def step(s, _):
        c0 = 4 * s
        for b in range(4):
            c = c0 + b
            ahead = c + 3

            @pl.when(ahead < _NCHUNK)
            def _(ahead=ahead, b=b):
                transfers(ahead, *ring[(b + 3) % 4])

            drain(*ring[b])
            compute(c, ring[b][0])
        return 0

    lax.fori_loop(0, _NCHUNK // 4, step, 0)
    for j in range(_NJ):
        pltpu.sync_copy(dots_v.at[j], out_hbm.at[pl.ds(j * _B + base, _BPW)])


def _tc_loss_body(d_ref, o_ref):
    d = d_ref[...]
    rows = lax.broadcasted_iota(jnp.int32, d.shape, 0)
    z = jnp.where(rows == 0, d, -d)
    logsig = -jnp.log(1.0 + jnp.exp(-z))
    o_ref[0, 0] = -jnp.sum(logsig) * (1.0 / _B)


def _tc_loss(dots):
    return pl.pallas_call(
        _tc_loss_body,
        out_shape=jax.ShapeDtypeStruct((1, 1), jnp.float32),
        in_specs=[pl.BlockSpec(memory_space=pltpu.VMEM)],
        out_specs=pl.BlockSpec(memory_space=pltpu.SMEM),
    )(dots)


def kernel(hidden, positives, table):
    negatives = jax.random.randint(
        jax.random.key(42), (_B, _NUM_NEG), 1, _VOCAB - 1, dtype=jnp.int32
    )
    idx = jnp.concatenate([positives[None, :], negatives.T], axis=0)
    dots = _sc_dots(hidden, idx.reshape(-1), table.reshape(1, _VOCAB, _E))
    return _tc_loss(dots.reshape(_NJ, _B))[0, 0]


# speedup vs baseline: 1.0061x; 1.0061x over previous
import functools

import jax
import jax.numpy as jnp
from jax import lax
from jax.experimental import pallas as pl
from jax.experimental.pallas import tpu as pltpu
from jax.experimental.pallas import tpu_sc as plsc

_B = 16384
_E = 64
_NUM_NEG = 5
_VOCAB = 1000000
_NJ = _NUM_NEG + 1
_NW = 32
_BPW = _B // _NW
_CHUNK = 32
_NCHUNK = _BPW // _CHUNK
_LANES = 16
_NG = _CHUNK // _LANES
_DUNROLL = 16
_BROWS = _NJ * _CHUNK + _CHUNK

_mesh = plsc.VectorSubcoreMesh(core_axis_name="c", subcore_axis_name="s")


@functools.partial(
    pl.kernel,
    mesh=_mesh,
    out_type=jax.ShapeDtypeStruct((_NJ * _B,), jnp.float32),
    scratch_types=[
        pltpu.VMEM((_NJ, _BPW), jnp.int32),
        pltpu.VMEM((_BROWS, _E), jnp.float32),
        pltpu.VMEM((_BROWS, _E), jnp.float32),
        pltpu.VMEM((_NJ, _BPW), jnp.float32),
        pltpu.SemaphoreType.DMA,
        pltpu.SemaphoreType.DMA,
    ],
    compiler_params=pltpu.CompilerParams(
        needs_layout_passes=False, use_tc_tiling_on_sc=True
    ),
)
def _sc_dots(hid_hbm, idx_hbm, table3_hbm, out_hbm,
             idx_v, buf_a, buf_b, dots_v, sem_a, sem_b):
    table_hbm = table3_hbm.at[0]
    wid = lax.axis_index("s") * 2 + lax.axis_index("c")
    base = wid * _BPW

    for j in range(_NJ):
        pltpu.sync_copy(idx_hbm.at[pl.ds(j * _B + base, _BPW)], idx_v.at[j])

    lane = lax.iota(jnp.int32, _LANES)
    buf_rows = [
        [lane + (j * _CHUNK + g * _LANES) for g in range(_NG)]
        for j in range(_NJ)
    ]
    hid_rows = [lane + (_NJ * _CHUNK + g * _LANES) for g in range(_NG)]
    zero = jnp.zeros((_LANES,), jnp.float32)

    def transfers(c, buf, sem):
        coff = c * _CHUNK
        for j in range(_NJ):
            for g in range(_NG):
                v = idx_v[j, pl.ds(coff + g * _LANES, _LANES)]
                for t in range(_LANES):
                    row = j * _CHUNK + g * _LANES + t
                    pltpu.async_copy(
                        table_hbm.at[pl.ds(v[t], 1), :],
                        buf.at[pl.ds(row, 1), :],
                        sem,
                    )
        pltpu.async_copy(
            hid_hbm.at[pl.ds(base + coff, _CHUNK), :],
            buf.at[pl.ds(_NJ * _CHUNK, _CHUNK), :],
            sem,
        )

    def drain(buf, sem):
        pltpu.make_async_copy(
            hid_hbm.at[pl.ds(0, _BROWS), :], buf, sem
        ).wait()

    def compute(c, buf):
        coff = c * _CHUNK
        for g in range(_NG):
            hrow = hid_rows[g]

            def dblk(k, accs, hrow=hrow, g=g, buf=buf):
                res = list(accs)
                d0 = k * _DUNROLL
                for t in range(_DUNROLL):
                    col = jnp.full((_LANES,), d0 + t, jnp.int32)
                    h = plsc.load_gather(buf, [hrow, col])
                    for j in range(_NJ):
                        e = plsc.load_gather(buf, [buf_rows[j][g], col])
                        res[j] = res[j] + e * h
                return tuple(res)

            accs = lax.fori_loop(0, _E // _DUNROLL, dblk, (zero,) * _NJ)
            for j in range(_NJ):
                dots_v[j, pl.ds(coff + g * _LANES, _LANES)] = accs[j]

    transfers(0, buf_a, sem_a)

    def step(s, _):
        c0 = 2 * s
        transfers(c0 + 1, buf_b, sem_b)
        drain(buf_a, sem_a)
        compute(c0, buf_a)

        @pl.when(s < _NCHUNK // 2 - 1)
        def _():
            transfers(c0 + 2, buf_a, sem_a)

        drain(buf_b, sem_b)
        compute(c0 + 1, buf_b)
        return 0

    lax.fori_loop(0, _NCHUNK // 2, step, 0)
    for j in range(_NJ):
        pltpu.sync_copy(dots_v.at[j], out_hbm.at[pl.ds(j * _B + base, _BPW)])


def _tc_loss_body(d_ref, o_ref):
    d = d_ref[...]
    rows = lax.broadcasted_iota(jnp.int32, d.shape, 0)
    z = jnp.where(rows == 0, d, -d)
    logsig = -jnp.log(1.0 + jnp.exp(-z))
    o_ref[0, 0] = -jnp.sum(logsig) * (1.0 / _B)


def _tc_loss(dots):
    return pl.pallas_call(
        _tc_loss_body,
        out_shape=jax.ShapeDtypeStruct((1, 1), jnp.float32),
        in_specs=[pl.BlockSpec(memory_space=pltpu.VMEM)],
        out_specs=pl.BlockSpec(memory_space=pltpu.SMEM),
    )(dots)


def kernel(hidden, positives, table):
    negatives = jax.random.randint(
        jax.random.key(42), (_B, _NUM_NEG), 1, _VOCAB - 1, dtype=jnp.int32
    )
    idx = jnp.concatenate([positives[None, :], negatives.T], axis=0)
    dots = _sc_dots(hidden, idx.reshape(-1), table.reshape(1, _VOCAB, _E))
    return _tc_loss(dots.reshape(_NJ, _B))[0, 0]


# speedup vs baseline: 1.0117x; 1.0056x over previous
import functools

import jax
import jax.numpy as jnp
from jax import lax
from jax.experimental import pallas as pl
from jax.experimental.pallas import tpu as pltpu
from jax.experimental.pallas import tpu_sc as plsc

_B = 16384
_E = 64
_NUM_NEG = 5
_VOCAB = 1000000
_NJ = _NUM_NEG + 1
_NW = 32
_BPW = _B // _NW
_CHUNK = 32
_NCHUNK = _BPW // _CHUNK
_LANES = 16
_NG = _CHUNK // _LANES
_DUNROLL = 16
_BROWS = _NJ * _CHUNK + _CHUNK

_mesh = plsc.VectorSubcoreMesh(core_axis_name="c", subcore_axis_name="s")


@functools.partial(
    pl.kernel,
    mesh=_mesh,
    out_type=jax.ShapeDtypeStruct((_NJ * _B,), jnp.float32),
    scratch_types=[
        pltpu.VMEM((_NJ, _BPW), jnp.int32),
        pltpu.VMEM((_BROWS, _E), jnp.float32),
        pltpu.VMEM((_BROWS, _E), jnp.float32),
        pltpu.VMEM((_NJ, _BPW), jnp.float32),
        pltpu.SemaphoreType.DMA,
        pltpu.SemaphoreType.DMA,
    ],
    compiler_params=pltpu.CompilerParams(
        needs_layout_passes=False, use_tc_tiling_on_sc=True
    ),
)
def _sc_dots(hid_hbm, idx_hbm, table3_hbm, out_hbm,
             idx_v, buf_a, buf_b, dots_v, sem_a, sem_b):
    table_hbm = table3_hbm.at[0]
    wid = lax.axis_index("s") * 2 + lax.axis_index("c")
    base = wid * _BPW

    idx_cps = [
        pltpu.async_copy(
            idx_hbm.at[pl.ds(j * _B + base, _BPW)], idx_v.at[j], sem_a
        )
        for j in range(_NJ)
    ]
    for cp in idx_cps:
        cp.wait()

    lane = lax.iota(jnp.int32, _LANES)
    buf_rows = [
        [lane + (j * _CHUNK + g * _LANES) for g in range(_NG)]
        for j in range(_NJ)
    ]
    hid_rows = [lane + (_NJ * _CHUNK + g * _LANES) for g in range(_NG)]
    zero = jnp.zeros((_LANES,), jnp.float32)

    def transfers(c, buf, sem):
        coff = c * _CHUNK
        for j in range(_NJ):
            for g in range(_NG):
                v = idx_v[j, pl.ds(coff + g * _LANES, _LANES)]
                for t in range(_LANES):
                    row = j * _CHUNK + g * _LANES + t
                    pltpu.async_copy(
                        table_hbm.at[pl.ds(v[t], 1), :],
                        buf.at[pl.ds(row, 1), :],
                        sem,
                    )
        pltpu.async_copy(
            hid_hbm.at[pl.ds(base + coff, _CHUNK), :],
            buf.at[pl.ds(_NJ * _CHUNK, _CHUNK), :],
            sem,
        )

    def drain(buf, sem):
        pltpu.make_async_copy(
            hid_hbm.at[pl.ds(0, _BROWS), :], buf, sem
        ).wait()

    def compute(c, buf):
        coff = c * _CHUNK
        for g in range(_NG):
            hrow = hid_rows[g]

            def dblk(k, accs, hrow=hrow, g=g, buf=buf):
                res = list(accs)
                d0 = k * _DUNROLL
                for t in range(_DUNROLL):
                    col = jnp.full((_LANES,), d0 + t, jnp.int32)
                    h = plsc.load_gather(buf, [hrow, col])
                    for j in range(_NJ):
                        e = plsc.load_gather(buf, [buf_rows[j][g], col])
                        res[j] = res[j] + e * h
                return tuple(res)

            accs = lax.fori_loop(0, _E // _DUNROLL, dblk, (zero,) * _NJ)
            for j in range(_NJ):
                dots_v[j, pl.ds(coff + g * _LANES, _LANES)] = accs[j]

    transfers(0, buf_a, sem_a)

    def step(s, _):
        c0 = 2 * s
        transfers(c0 + 1, buf_b, sem_b)
        drain(buf_a, sem_a)
        compute(c0, buf_a)

        @pl.when(s < _NCHUNK // 2 - 1)
        def _():
            transfers(c0 + 2, buf_a, sem_a)

        drain(buf_b, sem_b)
        compute(c0 + 1, buf_b)
        return 0

    lax.fori_loop(0, _NCHUNK // 2, step, 0)
    out_cps = [
        pltpu.async_copy(
            dots_v.at[j], out_hbm.at[pl.ds(j * _B + base, _BPW)], sem_a
        )
        for j in range(_NJ)
    ]
    for cp in out_cps:
        cp.wait()


def _tc_loss_body(d_ref, o_ref):
    d = d_ref[...]
    rows = lax.broadcasted_iota(jnp.int32, d.shape, 0)
    z = jnp.where(rows == 0, d, -d)
    logsig = -jnp.log(1.0 + jnp.exp(-z))
    o_ref[0, 0] = -jnp.sum(logsig) * (1.0 / _B)


def _tc_loss(dots):
    return pl.pallas_call(
        _tc_loss_body,
        out_shape=jax.ShapeDtypeStruct((1, 1), jnp.float32),
        in_specs=[pl.BlockSpec(memory_space=pltpu.VMEM)],
        out_specs=pl.BlockSpec(memory_space=pltpu.SMEM),
    )(dots)


def kernel(hidden, positives, table):
    negatives = jax.random.randint(
        jax.random.key(42), (_B, _NUM_NEG), 1, _VOCAB - 1, dtype=jnp.int32
    )
    idx = jnp.concatenate([positives[None, :], negatives.T], axis=0)
    dots = _sc_dots(hidden, idx.reshape(-1), table.reshape(1, _VOCAB, _E))
    return _tc_loss(dots.reshape(_NJ, _B))[0, 0]
